# Initial kernel scaffold; baseline (speedup 1.0000x reference)
import jax
import jax.numpy as jnp
from jax.experimental import pallas as pl


def kernel(x, edge_index, enc_W, enc_b, enc_g, enc_be, gcn_W, gcn_b, gat_W, gat_as, gat_ad, gat_b, rg_Wk, rg_Wq, rg_Wv, rg_Ws, rg_b, bn1_g, bn1_b, bn2_g, bn2_b, bn3_g, bn3_b, cls_W1, cls_b1, cls_W2, cls_b2):
    raise NotImplementedError("write your pallas kernel here")


# speedup vs baseline: 6.2125x; 6.2125x over previous
import functools

import jax
import jax.numpy as jnp
from jax import lax
from jax.experimental import pallas as pl
from jax.experimental.pallas import tpu as pltpu
from jax.experimental.pallas import tpu_sc as plsc

N = 10000
D = 128
E = 320000
HID = 64
C_OUT = 21

NC = 2
NS = 16
NW = NC * NS
NPAD = 10240
EPAD = 327680
TE = EPAD // NW
CH = 128
NCH = TE // CH
CHS = 64
NCHS = TE // CHS
RPT = NPAD // NS

_f32 = jnp.float32
_i32 = jnp.int32

_MESH = plsc.VectorSubcoreMesh(core_axis_name="c", subcore_axis_name="s",
                               num_cores=NC, num_subcores=NS)


def _wid():
    return lax.axis_index("c") * NS + lax.axis_index("s")


@functools.partial(
    pl.kernel,
    out_type=jax.ShapeDtypeStruct((NW, NPAD), _f32),
    mesh=_MESH,
    compiler_params=pltpu.CompilerParams(needs_layout_passes=False),
    scratch_types=[
        pltpu.VMEM((TE,), _i32),
        pltpu.VMEM((NPAD,), _f32),
    ],
)
def _sc_deg(dst_hbm, out_hbm, idxv, hist):
    wid = _wid()
    pltpu.sync_copy(dst_hbm.at[pl.ds(wid * TE, TE)], idxv)
    zeros16 = jnp.zeros((16,), _f32)
    ones16 = jnp.ones((16,), _f32)

    def zero_body(i, c):
        hist[pl.ds(i * 16, 16)] = zeros16
        return c

    lax.fori_loop(0, NPAD // 16, zero_body, 0)

    def body(g, c):
        di = idxv[pl.ds(g * 16, 16)]
        plsc.addupdate_scatter(hist, [di], ones16)
        return c

    lax.fori_loop(0, TE // 16, body, 0)
    pltpu.sync_copy(hist, out_hbm.at[wid])


@functools.partial(
    pl.kernel,
    out_type=jax.ShapeDtypeStruct((NC, NPAD, D), _f32),
    mesh=_MESH,
    compiler_params=pltpu.CompilerParams(needs_layout_passes=False),
    scratch_types=[
        pltpu.VMEM((CH,), _i32),
        pltpu.VMEM((CH,), _i32),
        pltpu.VMEM((CH, D), _f32),
        pltpu.VMEM_SHARED((NPAD, D), _f32),
    ],
)
def _sc_gcn(src_hbm, dst_hbm, tab_hbm, out_hbm, sidx, didx, rows, acc):
    cid = lax.axis_index("c")
    sid = lax.axis_index("s")
    wid = cid * NS + sid
    zeros16 = jnp.zeros((16,), _f32)

    def zero_body(i, c):
        for j in range(D // 16):
            rows[i, pl.ds(j * 16, 16)] = zeros16
        return c

    lax.fori_loop(0, CH, zero_body, 0)
    for b in range(RPT // CH):
        pltpu.sync_copy(rows, acc.at[pl.ds(sid * RPT + b * CH, CH)])
    plsc.subcore_barrier()

    base = wid * TE

    def chunk(c, carry):
        eb = base + c * CH
        pltpu.sync_copy(src_hbm.at[pl.ds(eb, CH)], sidx)
        pltpu.sync_copy(dst_hbm.at[pl.ds(eb, CH)], didx)
        pltpu.sync_copy(tab_hbm.at[sidx], rows)
        pltpu.sync_copy(rows, acc.at[didx], add=True)
        return carry

    lax.fori_loop(0, NCH, chunk, 0)
    plsc.subcore_barrier()
    pltpu.sync_copy(acc.at[pl.ds(sid * RPT, RPT)],
                    out_hbm.at[cid, pl.ds(sid * RPT, RPT)])


@functools.partial(
    pl.kernel,
    out_type=(
        jax.ShapeDtypeStruct((NC, NPAD, D), _f32),
        jax.ShapeDtypeStruct((NW, NPAD), _f32),
    ),
    mesh=_MESH,
    compiler_params=pltpu.CompilerParams(needs_layout_passes=False),
    scratch_types=[
        pltpu.VMEM((NPAD,), _f32),
        pltpu.VMEM((NPAD,), _f32),
        pltpu.VMEM((CHS,), _i32),
        pltpu.VMEM((CHS,), _i32),
        pltpu.VMEM((CHS, D), _f32),
        pltpu.VMEM((CHS,), _f32),
        pltpu.VMEM((NPAD,), _f32),
        pltpu.VMEM_SHARED((NPAD, D), _f32),
    ],
)
def _sc_gat(src_hbm, dst_hbm, tab_hbm, as_hbm, ad_hbm, num_hbm, den_hbm,
            asv, adv, sidx, didx, rows, eeb, denh, acc):
    cid = lax.axis_index("c")
    sid = lax.axis_index("s")
    wid = cid * NS + sid
    zeros16 = jnp.zeros((16,), _f32)

    pltpu.sync_copy(as_hbm, asv)
    pltpu.sync_copy(ad_hbm, adv)

    def zero_hist(i, c):
        denh[pl.ds(i * 16, 16)] = zeros16
        return c

    lax.fori_loop(0, NPAD // 16, zero_hist, 0)

    def zero_body(i, c):
        for j in range(D // 16):
            rows[i, pl.ds(j * 16, 16)] = zeros16
        return c

    lax.fori_loop(0, CHS, zero_body, 0)
    for b in range(RPT // CHS):
        pltpu.sync_copy(rows, acc.at[pl.ds(sid * RPT + b * CHS, CHS)])
    plsc.subcore_barrier()

    base = wid * TE

    def chunk(c, carry):
        eb = base + c * CHS
        pltpu.sync_copy(src_hbm.at[pl.ds(eb, CHS)], sidx)
        pltpu.sync_copy(dst_hbm.at[pl.ds(eb, CHS)], didx)
        pltpu.sync_copy(tab_hbm.at[sidx], rows)
        for g in range(CHS // 16):
            si = sidx[pl.ds(g * 16, 16)]
            di = didx[pl.ds(g * 16, 16)]
            e = plsc.load_gather(asv, [si]) + plsc.load_gather(adv, [di])
            e = jnp.where(e > 0.0, e, e * 0.2)
            ee = jnp.exp(e)
            eeb[pl.ds(g * 16, 16)] = ee
            plsc.addupdate_scatter(denh, [di], ee)

        def row(i, c2):
            s = plsc.load_gather(eeb, [jnp.zeros((16,), _i32) + i])
            for j in range(D // 16):
                rows[i, pl.ds(j * 16, 16)] = rows[i, pl.ds(j * 16, 16)] * s
            return c2

        lax.fori_loop(0, CHS, row, 0)
        pltpu.sync_copy(rows, acc.at[didx], add=True)
        return carry

    lax.fori_loop(0, NCHS, chunk, 0)
    plsc.subcore_barrier()
    pltpu.sync_copy(acc.at[pl.ds(sid * RPT, RPT)],
                    num_hbm.at[cid, pl.ds(sid * RPT, RPT)])
    pltpu.sync_copy(denh, den_hbm.at[wid])


@functools.partial(
    pl.kernel,
    out_type=jax.ShapeDtypeStruct((NC, NPAD, D), _f32),
    mesh=_MESH,
    compiler_params=pltpu.CompilerParams(needs_layout_passes=False),
    scratch_types=[
        pltpu.VMEM((CHS,), _i32),
        pltpu.VMEM((CHS,), _i32),
        pltpu.VMEM((CHS, D), _f32),
        pltpu.VMEM((CHS, D), _f32),
        pltpu.VMEM((CHS, D), _f32),
        pltpu.VMEM_SHARED((NPAD, D), _f32),
    ],
)
def _sc_rg(src_hbm, dst_hbm, k_hbm, q_hbm, v_hbm, out_hbm,
           sidx, didx, kr, qr, vr, acc):
    cid = lax.axis_index("c")
    sid = lax.axis_index("s")
    wid = cid * NS + sid
    zeros16 = jnp.zeros((16,), _f32)
    ones16 = jnp.ones((16,), _f32)

    def zero_body(i, c):
        for j in range(D // 16):
            kr[i, pl.ds(j * 16, 16)] = zeros16
        return c

    lax.fori_loop(0, CHS, zero_body, 0)
    for b in range(RPT // CHS):
        pltpu.sync_copy(kr, acc.at[pl.ds(sid * RPT + b * CHS, CHS)])
    plsc.subcore_barrier()

    base = wid * TE

    def chunk(c, carry):
        eb = base + c * CHS
        pltpu.sync_copy(src_hbm.at[pl.ds(eb, CHS)], sidx)
        pltpu.sync_copy(dst_hbm.at[pl.ds(eb, CHS)], didx)
        pltpu.sync_copy(k_hbm.at[didx], kr)
        pltpu.sync_copy(q_hbm.at[sidx], qr)
        pltpu.sync_copy(v_hbm.at[sidx], vr)

        def row(i, c2):
            for j in range(D // 16):
                sl = pl.ds(j * 16, 16)
                t = kr[i, sl] + qr[i, sl]
                kr[i, sl] = vr[i, sl] / (ones16 + jnp.exp(-t))
            return c2

        lax.fori_loop(0, CHS, row, 0)
        pltpu.sync_copy(kr, acc.at[didx], add=True)
        return carry

    lax.fori_loop(0, NCHS, chunk, 0)
    plsc.subcore_barrier()
    pltpu.sync_copy(acc.at[pl.ds(sid * RPT, RPT)],
                    out_hbm.at[cid, pl.ds(sid * RPT, RPT)])


def _bn_rows(h, g, b):
    m = jnp.mean(h, axis=0, keepdims=True)
    v = jnp.mean((h - m) * (h - m), axis=0, keepdims=True)
    return (h - m) / jnp.sqrt(v + 1e-5) * g + b


def _dot(a, b):
    return jnp.dot(a, b, preferred_element_type=_f32)


def _tc_enc_body(degp, x, eW, eb, eg, ebe, gW, hlp, selfc, dinv_o):
    deg = lax.dot_general(degp[...], jnp.ones((NW, 1), _f32),
                          (((0,), (0,)), ((), ())))[:N] + 1.0
    dinv = lax.rsqrt(deg)
    h = _dot(x[...], eW[...]) + eb[...]
    h = _bn_rows(h, eg[...], ebe[...])
    h = jnp.maximum(h, 0.0)
    hl = _dot(h, gW[...])
    hlp[0:N, :] = hl * dinv
    hlp[N:NPAD, :] = jnp.zeros((NPAD - N, D), _f32)
    selfc[...] = hl * (dinv * dinv)
    dinv_o[...] = dinv


_tc_enc = pl.pallas_call(
    _tc_enc_body,
    out_shape=[
        jax.ShapeDtypeStruct((NPAD, D), _f32),
        jax.ShapeDtypeStruct((N, D), _f32),
        jax.ShapeDtypeStruct((N, 1), _f32),
    ],
)


def _tc_gat_prep_body(p, selfc, dinv, gb, g1, b1, gW, gas, gad,
                      hgp, asp, adp, x1_o, snum, sden):
    agg = (p[0, 0:N, :] + p[1, 0:N, :]) * dinv[...] + selfc[...] + gb[...]
    x1 = jnp.maximum(agg, 0.0)
    x1 = _bn_rows(x1, g1[...], b1[...])
    hg = _dot(x1, gW[...])
    a_s = _dot(hg, gas[...])
    a_d = _dot(hg, gad[...])
    es = a_s + a_d
    es = jnp.where(es > 0.0, es, es * 0.2)
    ee = jnp.exp(es)
    hgp[0:N, :] = hg
    hgp[N:NPAD, :] = jnp.zeros((NPAD - N, D), _f32)
    asp[0:N, :] = a_s
    asp[N:NPAD, :] = jnp.zeros((NPAD - N, 1), _f32)
    adp[0:N, :] = a_d
    adp[N:NPAD, :] = jnp.zeros((NPAD - N, 1), _f32)
    x1_o[...] = x1
    snum[...] = hg * ee
    sden[...] = ee


_tc_gat_prep = pl.pallas_call(
    _tc_gat_prep_body,
    out_shape=[
        jax.ShapeDtypeStruct((NPAD, D), _f32),
        jax.ShapeDtypeStruct((NPAD, 1), _f32),
        jax.ShapeDtypeStruct((NPAD, 1), _f32),
        jax.ShapeDtypeStruct((N, D), _f32),
        jax.ShapeDtypeStruct((N, D), _f32),
        jax.ShapeDtypeStruct((N, 1), _f32),
    ],
)


def _tc_rg_prep_body(p, denp, snum, sden, x1, gatb, g2, b2, Wk, Wq, Wv, Ws,
                     kp, qp, vp, skip_o, x2_o):
    num = p[0, 0:N, :] + p[1, 0:N, :] + snum[...]
    den = lax.dot_general(denp[...], jnp.ones((NW, 1), _f32),
                          (((0,), (0,)), ((), ())))[:N] + sden[...]
    x2 = num / (den + 1e-16) + gatb[...]
    x2 = jnp.maximum(x2, 0.0)
    x2 = _bn_rows(x2, g2[...], b2[...])
    x2 = x1[...] + x2
    zpad = jnp.zeros((NPAD - N, D), _f32)
    kp[0:N, :] = _dot(x2, Wk[...])
    kp[N:NPAD, :] = zpad
    qp[0:N, :] = _dot(x2, Wq[...])
    qp[N:NPAD, :] = zpad
    vp[0:N, :] = _dot(x2, Wv[...])
    vp[N:NPAD, :] = zpad
    skip_o[...] = _dot(x2, Ws[...])
    x2_o[...] = x2


_tc_rg_prep = pl.pallas_call(
    _tc_rg_prep_body,
    out_shape=[
        jax.ShapeDtypeStruct((NPAD, D), _f32),
        jax.ShapeDtypeStruct((NPAD, D), _f32),
        jax.ShapeDtypeStruct((NPAD, D), _f32),
        jax.ShapeDtypeStruct((N, D), _f32),
        jax.ShapeDtypeStruct((N, D), _f32),
    ],
)


def _tc_cls_body(p, skip, x2, rgb, g3, b3, W1, c1, W2, c2, out):
    x3 = p[0, 0:N, :] + p[1, 0:N, :] + skip[...] + rgb[...]
    x3 = jnp.maximum(x3, 0.0)
    x3 = _bn_rows(x3, g3[...], b3[...])
    x3 = x2[...] + x3
    z = jnp.maximum(_dot(x3, W1[...]) + c1[...], 0.0)
    out[...] = _dot(z, W2[...]) + c2[...]


_tc_cls = pl.pallas_call(
    _tc_cls_body,
    out_shape=jax.ShapeDtypeStruct((N, C_OUT), _f32),
)


def kernel(x, edge_index, enc_W, enc_b, enc_g, enc_be, gcn_W, gcn_b,
           gat_W, gat_as, gat_ad, gat_b, rg_Wk, rg_Wq, rg_Wv, rg_Ws, rg_b,
           bn1_g, bn1_b, bn2_g, bn2_b, bn3_g, bn3_b,
           cls_W1, cls_b1, cls_W2, cls_b2):
    pad = jnp.full((EPAD - E,), N, _i32)
    srcp = jnp.concatenate([edge_index[0].astype(_i32), pad])
    dstp = jnp.concatenate([edge_index[1].astype(_i32), pad])

    degp = _sc_deg(dstp)

    hlp, selfc, dinv = _tc_enc(
        degp, x, enc_W, enc_b.reshape(1, D), enc_g.reshape(1, D),
        enc_be.reshape(1, D), gcn_W)

    gcn_parts = _sc_gcn(srcp, dstp, hlp)

    hgp, asp, adp, x1, snum, sden = _tc_gat_prep(
        gcn_parts, selfc, dinv, gcn_b.reshape(1, D), bn1_g.reshape(1, D),
        bn1_b.reshape(1, D), gat_W, gat_as.reshape(D, 1), gat_ad.reshape(D, 1))

    gat_num, gat_den = _sc_gat(srcp, dstp, hgp, asp.reshape(NPAD),
                               adp.reshape(NPAD))

    kp, qp, vp, skip, x2 = _tc_rg_prep(
        gat_num, gat_den, snum, sden, x1, gat_b.reshape(1, D), bn2_g.reshape(1, D),
        bn2_b.reshape(1, D), rg_Wk, rg_Wq, rg_Wv, rg_Ws)

    rg_parts = _sc_rg(srcp, dstp, kp, qp, vp)

    logits = _tc_cls(
        rg_parts, skip, x2, rg_b.reshape(1, D), bn3_g.reshape(1, D),
        bn3_b.reshape(1, D), cls_W1, cls_b1.reshape(1, HID), cls_W2,
        cls_b2.reshape(1, C_OUT))
    return logits


# speedup vs baseline: 7.7627x; 1.2495x over previous
import functools

import jax
import jax.numpy as jnp
from jax import lax
from jax.experimental import pallas as pl
from jax.experimental.pallas import tpu as pltpu
from jax.experimental.pallas import tpu_sc as plsc

N = 10000
D = 128
E = 320000
HID = 64
C_OUT = 21

NC = 2
NS = 16
NW = NC * NS
NPAD = 10240
EPAD = 327680
TEP = EPAD // NS
TE0 = 13824
TE1 = TEP - TE0
CH = 128
CHS = 64
RPT = NPAD // NS

_f32 = jnp.float32
_i32 = jnp.int32

_MESH = plsc.VectorSubcoreMesh(core_axis_name="c", subcore_axis_name="s",
                               num_cores=NC, num_subcores=NS)


def _wid():
    return lax.axis_index("c") * NS + lax.axis_index("s")


@functools.partial(
    pl.kernel,
    out_type=jax.ShapeDtypeStruct((NW, NPAD), _f32),
    mesh=_MESH,
    compiler_params=pltpu.CompilerParams(needs_layout_passes=False),
    scratch_types=[
        pltpu.VMEM((TE0,), _i32),
        pltpu.VMEM((NPAD,), _f32),
    ],
)
def _sc_deg(dst_hbm, out_hbm, idxv, hist):
    cid = lax.axis_index("c")
    sid = lax.axis_index("s")
    wid = cid * NS + sid
    base = sid * TEP + cid * TE0

    @pl.when(cid == 0)
    def _():
        pltpu.sync_copy(dst_hbm.at[pl.ds(base, TE0)], idxv.at[pl.ds(0, TE0)])

    @pl.when(cid == 1)
    def _():
        pltpu.sync_copy(dst_hbm.at[pl.ds(base, TE1)], idxv.at[pl.ds(0, TE1)])

    zeros16 = jnp.zeros((16,), _f32)
    ones16 = jnp.ones((16,), _f32)

    def zero_body(i, c):
        hist[pl.ds(i * 16, 16)] = zeros16
        return c

    lax.fori_loop(0, NPAD // 16, zero_body, 0)

    def body(g, c):
        di = idxv[pl.ds(g * 16, 16)]
        plsc.addupdate_scatter(hist, [di], ones16)
        return c

    n16 = jnp.where(cid == 0, TE0 // 16, TE1 // 16)
    lax.fori_loop(0, n16, body, 0)
    pltpu.sync_copy(hist, out_hbm.at[wid])


@functools.partial(
    pl.kernel,
    out_type=jax.ShapeDtypeStruct((NC, NPAD, D), _f32),
    mesh=_MESH,
    compiler_params=pltpu.CompilerParams(needs_layout_passes=False),
    scratch_types=[
        pltpu.VMEM((CH,), _i32),
        pltpu.VMEM((CH,), _i32),
        pltpu.VMEM((CH, D), _f32),
        pltpu.VMEM_SHARED((NPAD, D), _f32),
    ],
)
def _sc_gcn(src_hbm, dst_hbm, tab_hbm, out_hbm, sidx, didx, rows, acc):
    cid = lax.axis_index("c")
    sid = lax.axis_index("s")
    wid = cid * NS + sid
    zeros16 = jnp.zeros((16,), _f32)

    def zero_body(i, c):
        for j in range(D // 16):
            rows[i, pl.ds(j * 16, 16)] = zeros16
        return c

    lax.fori_loop(0, CH, zero_body, 0)
    for b in range(RPT // CH):
        pltpu.sync_copy(rows, acc.at[pl.ds(sid * RPT + b * CH, CH)])
    plsc.subcore_barrier()

    base = sid * TEP + cid * TE0

    def chunk(c, carry):
        eb = base + c * CH
        pltpu.sync_copy(src_hbm.at[pl.ds(eb, CH)], sidx)
        pltpu.sync_copy(dst_hbm.at[pl.ds(eb, CH)], didx)
        pltpu.sync_copy(tab_hbm.at[sidx], rows)
        pltpu.sync_copy(rows, acc.at[didx], add=True)
        return carry

    nch = jnp.where(cid == 0, TE0 // CH, TE1 // CH)
    lax.fori_loop(0, nch, chunk, 0)
    plsc.subcore_barrier()
    pltpu.sync_copy(acc.at[pl.ds(sid * RPT, RPT)],
                    out_hbm.at[cid, pl.ds(sid * RPT, RPT)])


@functools.partial(
    pl.kernel,
    out_type=(
        jax.ShapeDtypeStruct((NC, NPAD, D), _f32),
        jax.ShapeDtypeStruct((NW, NPAD), _f32),
    ),
    mesh=_MESH,
    compiler_params=pltpu.CompilerParams(needs_layout_passes=False),
    scratch_types=[
        pltpu.VMEM((NPAD,), _f32),
        pltpu.VMEM((NPAD,), _f32),
        pltpu.VMEM((CHS,), _i32),
        pltpu.VMEM((CHS,), _i32),
        pltpu.VMEM((CHS, D), _f32),
        pltpu.VMEM((CHS,), _f32),
        pltpu.VMEM((NPAD,), _f32),
        pltpu.VMEM_SHARED((NPAD, D), _f32),
    ],
)
def _sc_gat(src_hbm, dst_hbm, tab_hbm, as_hbm, ad_hbm, num_hbm, den_hbm,
            asv, adv, sidx, didx, rows, eeb, denh, acc):
    cid = lax.axis_index("c")
    sid = lax.axis_index("s")
    wid = cid * NS + sid
    zeros16 = jnp.zeros((16,), _f32)

    pltpu.sync_copy(as_hbm, asv)
    pltpu.sync_copy(ad_hbm, adv)

    def zero_hist(i, c):
        denh[pl.ds(i * 16, 16)] = zeros16
        return c

    lax.fori_loop(0, NPAD // 16, zero_hist, 0)

    def zero_body(i, c):
        for j in range(D // 16):
            rows[i, pl.ds(j * 16, 16)] = zeros16
        return c

    lax.fori_loop(0, CHS, zero_body, 0)
    for b in range(RPT // CHS):
        pltpu.sync_copy(rows, acc.at[pl.ds(sid * RPT + b * CHS, CHS)])
    plsc.subcore_barrier()

    base = sid * TEP + cid * TE0

    def chunk(c, carry):
        eb = base + c * CHS
        pltpu.sync_copy(src_hbm.at[pl.ds(eb, CHS)], sidx)
        pltpu.sync_copy(dst_hbm.at[pl.ds(eb, CHS)], didx)
        pltpu.sync_copy(tab_hbm.at[sidx], rows)
        for g in range(CHS // 16):
            si = sidx[pl.ds(g * 16, 16)]
            di = didx[pl.ds(g * 16, 16)]
            e = plsc.load_gather(asv, [si]) + plsc.load_gather(adv, [di])
            e = jnp.where(e > 0.0, e, e * 0.2)
            ee = jnp.exp(e)
            eeb[pl.ds(g * 16, 16)] = ee
            plsc.addupdate_scatter(denh, [di], ee)

        def row(i, c2):
            s = plsc.load_gather(eeb, [jnp.zeros((16,), _i32) + i])
            for j in range(D // 16):
                rows[i, pl.ds(j * 16, 16)] = rows[i, pl.ds(j * 16, 16)] * s
            return c2

        lax.fori_loop(0, CHS, row, 0)
        pltpu.sync_copy(rows, acc.at[didx], add=True)
        return carry

    nch = jnp.where(cid == 0, TE0 // CHS, TE1 // CHS)
    lax.fori_loop(0, nch, chunk, 0)
    plsc.subcore_barrier()
    pltpu.sync_copy(acc.at[pl.ds(sid * RPT, RPT)],
                    num_hbm.at[cid, pl.ds(sid * RPT, RPT)])
    pltpu.sync_copy(denh, den_hbm.at[wid])


@functools.partial(
    pl.kernel,
    out_type=jax.ShapeDtypeStruct((NC, NPAD, D), _f32),
    mesh=_MESH,
    compiler_params=pltpu.CompilerParams(needs_layout_passes=False),
    scratch_types=[
        pltpu.VMEM((CHS,), _i32),
        pltpu.VMEM((CHS,), _i32),
        pltpu.VMEM((CHS, D), _f32),
        pltpu.VMEM((CHS, D), _f32),
        pltpu.VMEM((CHS, D), _f32),
        pltpu.VMEM_SHARED((NPAD, D), _f32),
    ],
)
def _sc_rg(src_hbm, dst_hbm, k_hbm, q_hbm, v_hbm, out_hbm,
           sidx, didx, kr, qr, vr, acc):
    cid = lax.axis_index("c")
    sid = lax.axis_index("s")
    wid = cid * NS + sid
    zeros16 = jnp.zeros((16,), _f32)
    ones16 = jnp.ones((16,), _f32)

    def zero_body(i, c):
        for j in range(D // 16):
            kr[i, pl.ds(j * 16, 16)] = zeros16
        return c

    lax.fori_loop(0, CHS, zero_body, 0)
    for b in range(RPT // CHS):
        pltpu.sync_copy(kr, acc.at[pl.ds(sid * RPT + b * CHS, CHS)])
    plsc.subcore_barrier()

    base = sid * TEP + cid * TE0

    def chunk(c, carry):
        eb = base + c * CHS
        pltpu.sync_copy(src_hbm.at[pl.ds(eb, CHS)], sidx)
        pltpu.sync_copy(dst_hbm.at[pl.ds(eb, CHS)], didx)
        pltpu.sync_copy(k_hbm.at[didx], kr)
        pltpu.sync_copy(q_hbm.at[sidx], qr)
        pltpu.sync_copy(v_hbm.at[sidx], vr)

        def row(i, c2):
            for j in range(D // 16):
                sl = pl.ds(j * 16, 16)
                t = kr[i, sl] + qr[i, sl]
                kr[i, sl] = vr[i, sl] / (ones16 + jnp.exp(-t))
            return c2

        lax.fori_loop(0, CHS, row, 0)
        pltpu.sync_copy(kr, acc.at[didx], add=True)
        return carry

    nch = jnp.where(cid == 0, TE0 // CHS, TE1 // CHS)
    lax.fori_loop(0, nch, chunk, 0)
    plsc.subcore_barrier()
    pltpu.sync_copy(acc.at[pl.ds(sid * RPT, RPT)],
                    out_hbm.at[cid, pl.ds(sid * RPT, RPT)])


def _bn_rows(h, g, b):
    m = jnp.mean(h, axis=0, keepdims=True)
    v = jnp.mean((h - m) * (h - m), axis=0, keepdims=True)
    return (h - m) / jnp.sqrt(v + 1e-5) * g + b


def _dot(a, b):
    return jnp.dot(a, b, preferred_element_type=_f32)


def _tc_enc_body(degp, x, eW, eb, eg, ebe, gW, hlp, selfc, dinv_o):
    deg = lax.dot_general(degp[...], jnp.ones((NW, 1), _f32),
                          (((0,), (0,)), ((), ())))[:N] + 1.0
    dinv = lax.rsqrt(deg)
    h = _dot(x[...], eW[...]) + eb[...]
    h = _bn_rows(h, eg[...], ebe[...])
    h = jnp.maximum(h, 0.0)
    hl = _dot(h, gW[...])
    hlp[0:N, :] = hl * dinv
    hlp[N:NPAD, :] = jnp.zeros((NPAD - N, D), _f32)
    selfc[...] = hl * (dinv * dinv)
    dinv_o[...] = dinv


_tc_enc = pl.pallas_call(
    _tc_enc_body,
    out_shape=[
        jax.ShapeDtypeStruct((NPAD, D), _f32),
        jax.ShapeDtypeStruct((N, D), _f32),
        jax.ShapeDtypeStruct((N, 1), _f32),
    ],
)


def _tc_gat_prep_body(p, selfc, dinv, gb, g1, b1, gW, gas, gad,
                      hgp, asp, adp, x1_o, snum, sden):
    agg = (p[0, 0:N, :] + p[1, 0:N, :]) * dinv[...] + selfc[...] + gb[...]
    x1 = jnp.maximum(agg, 0.0)
    x1 = _bn_rows(x1, g1[...], b1[...])
    hg = _dot(x1, gW[...])
    a_s = _dot(hg, gas[...])
    a_d = _dot(hg, gad[...])
    es = a_s + a_d
    es = jnp.where(es > 0.0, es, es * 0.2)
    ee = jnp.exp(es)
    hgp[0:N, :] = hg
    hgp[N:NPAD, :] = jnp.zeros((NPAD - N, D), _f32)
    asp[0:N, :] = a_s
    asp[N:NPAD, :] = jnp.zeros((NPAD - N, 1), _f32)
    adp[0:N, :] = a_d
    adp[N:NPAD, :] = jnp.zeros((NPAD - N, 1), _f32)
    x1_o[...] = x1
    snum[...] = hg * ee
    sden[...] = ee


_tc_gat_prep = pl.pallas_call(
    _tc_gat_prep_body,
    out_shape=[
        jax.ShapeDtypeStruct((NPAD, D), _f32),
        jax.ShapeDtypeStruct((NPAD, 1), _f32),
        jax.ShapeDtypeStruct((NPAD, 1), _f32),
        jax.ShapeDtypeStruct((N, D), _f32),
        jax.ShapeDtypeStruct((N, D), _f32),
        jax.ShapeDtypeStruct((N, 1), _f32),
    ],
)


def _tc_rg_prep_body(p, denp, snum, sden, x1, gatb, g2, b2, Wk, Wq, Wv, Ws,
                     kp, qp, vp, skip_o, x2_o):
    num = p[0, 0:N, :] + p[1, 0:N, :] + snum[...]
    den = lax.dot_general(denp[...], jnp.ones((NW, 1), _f32),
                          (((0,), (0,)), ((), ())))[:N] + sden[...]
    x2 = num / (den + 1e-16) + gatb[...]
    x2 = jnp.maximum(x2, 0.0)
    x2 = _bn_rows(x2, g2[...], b2[...])
    x2 = x1[...] + x2
    zpad = jnp.zeros((NPAD - N, D), _f32)
    kp[0:N, :] = _dot(x2, Wk[...])
    kp[N:NPAD, :] = zpad
    qp[0:N, :] = _dot(x2, Wq[...])
    qp[N:NPAD, :] = zpad
    vp[0:N, :] = _dot(x2, Wv[...])
    vp[N:NPAD, :] = zpad
    skip_o[...] = _dot(x2, Ws[...])
    x2_o[...] = x2


_tc_rg_prep = pl.pallas_call(
    _tc_rg_prep_body,
    out_shape=[
        jax.ShapeDtypeStruct((NPAD, D), _f32),
        jax.ShapeDtypeStruct((NPAD, D), _f32),
        jax.ShapeDtypeStruct((NPAD, D), _f32),
        jax.ShapeDtypeStruct((N, D), _f32),
        jax.ShapeDtypeStruct((N, D), _f32),
    ],
)


def _tc_cls_body(p, skip, x2, rgb, g3, b3, W1, c1, W2, c2, out):
    x3 = p[0, 0:N, :] + p[1, 0:N, :] + skip[...] + rgb[...]
    x3 = jnp.maximum(x3, 0.0)
    x3 = _bn_rows(x3, g3[...], b3[...])
    x3 = x2[...] + x3
    z = jnp.maximum(_dot(x3, W1[...]) + c1[...], 0.0)
    out[...] = _dot(z, W2[...]) + c2[...]


_tc_cls = pl.pallas_call(
    _tc_cls_body,
    out_shape=jax.ShapeDtypeStruct((N, C_OUT), _f32),
)


def kernel(x, edge_index, enc_W, enc_b, enc_g, enc_be, gcn_W, gcn_b,
           gat_W, gat_as, gat_ad, gat_b, rg_Wk, rg_Wq, rg_Wv, rg_Ws, rg_b,
           bn1_g, bn1_b, bn2_g, bn2_b, bn3_g, bn3_b,
           cls_W1, cls_b1, cls_W2, cls_b2):
    pad = jnp.full((EPAD - E,), N, _i32)
    srcp = jnp.concatenate([edge_index[0].astype(_i32), pad])
    dstp = jnp.concatenate([edge_index[1].astype(_i32), pad])

    degp = _sc_deg(dstp)

    hlp, selfc, dinv = _tc_enc(
        degp, x, enc_W, enc_b.reshape(1, D), enc_g.reshape(1, D),
        enc_be.reshape(1, D), gcn_W)

    gcn_parts = _sc_gcn(srcp, dstp, hlp)

    hgp, asp, adp, x1, snum, sden = _tc_gat_prep(
        gcn_parts, selfc, dinv, gcn_b.reshape(1, D), bn1_g.reshape(1, D),
        bn1_b.reshape(1, D), gat_W, gat_as.reshape(D, 1), gat_ad.reshape(D, 1))

    gat_num, gat_den = _sc_gat(srcp, dstp, hgp, asp.reshape(NPAD),
                               adp.reshape(NPAD))

    kp, qp, vp, skip, x2 = _tc_rg_prep(
        gat_num, gat_den, snum, sden, x1, gat_b.reshape(1, D), bn2_g.reshape(1, D),
        bn2_b.reshape(1, D), rg_Wk, rg_Wq, rg_Wv, rg_Ws)

    rg_parts = _sc_rg(srcp, dstp, kp, qp, vp)

    logits = _tc_cls(
        rg_parts, skip, x2, rg_b.reshape(1, D), bn3_g.reshape(1, D),
        bn3_b.reshape(1, D), cls_W1, cls_b1.reshape(1, HID), cls_W2,
        cls_b2.reshape(1, C_OUT))
    return logits


# speedup vs baseline: 13.4131x; 1.7279x over previous
import functools

import jax
import jax.numpy as jnp
from jax import lax
from jax.experimental import pallas as pl
from jax.experimental.pallas import tpu as pltpu
from jax.experimental.pallas import tpu_sc as plsc

N = 10000
D = 128
E = 320000
HID = 64
C_OUT = 21

NC = 2
NS = 16
NW = NC * NS
NPAD = 10240
EPAD = 327680
TEP = EPAD // NS
TE0 = 13824
TE1 = TEP - TE0
CH = 128
CHS = 64
NACC = 10112
RPT = NACC // NS
EPADX = EPAD + CH

_f32 = jnp.float32
_i32 = jnp.int32

_MESH = plsc.VectorSubcoreMesh(core_axis_name="c", subcore_axis_name="s",
                               num_cores=NC, num_subcores=NS)
_SC_PARAMS = pltpu.CompilerParams(needs_layout_passes=False)


def _zero_acc(zbuf, acc, sid, nrows):
    off = sid * RPT
    done = 0
    while done < RPT:
        step = min(nrows, RPT - done)
        pltpu.sync_copy(zbuf.at[pl.ds(0, step)],
                        acc.at[pl.ds(off + done, step)])
        done += step


@functools.partial(
    pl.kernel,
    out_type=jax.ShapeDtypeStruct((NW, NACC), _f32),
    mesh=_MESH,
    compiler_params=_SC_PARAMS,
    scratch_types=[
        pltpu.VMEM((TE0,), _i32),
        pltpu.VMEM((NACC,), _f32),
    ],
)
def _sc_deg(dst_hbm, out_hbm, idxv, hist):
    cid = lax.axis_index("c")
    sid = lax.axis_index("s")
    wid = cid * NS + sid
    base = sid * TEP + cid * TE0

    @pl.when(cid == 0)
    def _():
        pltpu.sync_copy(dst_hbm.at[pl.ds(base, TE0)], idxv.at[pl.ds(0, TE0)])

    @pl.when(cid == 1)
    def _():
        pltpu.sync_copy(dst_hbm.at[pl.ds(base, TE1)], idxv.at[pl.ds(0, TE1)])

    zeros16 = jnp.zeros((16,), _f32)
    ones16 = jnp.ones((16,), _f32)

    def zero_body(i, c):
        hist[pl.ds(i * 16, 16)] = zeros16
        return c

    lax.fori_loop(0, NACC // 16, zero_body, 0)

    def body(g, c):
        di = idxv[pl.ds(g * 16, 16)]
        plsc.addupdate_scatter(hist, [di], ones16)
        return c

    n16 = jnp.where(cid == 0, TE0 // 16, TE1 // 16)
    lax.fori_loop(0, n16, body, 0)
    pltpu.sync_copy(hist, out_hbm.at[wid])


@functools.partial(
    pl.kernel,
    out_type=jax.ShapeDtypeStruct((NC, NACC, D), _f32),
    mesh=_MESH,
    compiler_params=_SC_PARAMS,
    scratch_types=[
        pltpu.VMEM((CH,), _i32),
        pltpu.VMEM((CH,), _i32),
        pltpu.VMEM((CH,), _i32),
        pltpu.VMEM((CH,), _i32),
        pltpu.VMEM((CH, D), _f32),
        pltpu.VMEM((CH, D), _f32),
        pltpu.VMEM_SHARED((NACC, D), _f32),
        pltpu.SemaphoreType.DMA,
        pltpu.SemaphoreType.DMA,
    ],
)
def _sc_gcn(src_hbm, dst_hbm, tab_hbm, out_hbm,
            sidx0, didx0, sidx1, didx1, rows0, rows1, acc, g0, g1):
    cid = lax.axis_index("c")
    sid = lax.axis_index("s")
    zeros16 = jnp.zeros((16,), _f32)

    def zero_body(i, c):
        for j in range(D // 16):
            rows0[i, pl.ds(j * 16, 16)] = zeros16
        return c

    lax.fori_loop(0, CH, zero_body, 0)
    _zero_acc(rows0, acc, sid, CH)
    plsc.subcore_barrier()

    base = sid * TEP + cid * TE0
    nch = jnp.where(cid == 0, TE0 // CH, TE1 // CH)

    def load_idx(si, di, c):
        eb = base + c * CH
        pltpu.sync_copy(src_hbm.at[pl.ds(eb, CH)], si)
        pltpu.sync_copy(dst_hbm.at[pl.ds(eb, CH)], di)

    load_idx(sidx0, didx0, 0)
    pltpu.async_copy(tab_hbm.at[sidx0], rows0, g0)

    def body(cc, carry):
        c1 = 2 * cc + 1
        load_idx(sidx1, didx1, c1)
        pltpu.async_copy(tab_hbm.at[sidx1], rows1, g1)
        pltpu.make_async_copy(tab_hbm.at[pl.ds(0, CH)], rows0, g0).wait()
        pltpu.sync_copy(rows0, acc.at[didx0], add=True)

        @pl.when(2 * cc + 2 < nch)
        def _():
            load_idx(sidx0, didx0, 2 * cc + 2)
            pltpu.async_copy(tab_hbm.at[sidx0], rows0, g0)

        pltpu.make_async_copy(tab_hbm.at[pl.ds(0, CH)], rows1, g1).wait()
        pltpu.sync_copy(rows1, acc.at[didx1], add=True)
        return carry

    lax.fori_loop(0, nch // 2, body, 0)
    plsc.subcore_barrier()
    pltpu.sync_copy(acc.at[pl.ds(sid * RPT, RPT)],
                    out_hbm.at[cid, pl.ds(sid * RPT, RPT)])


@functools.partial(
    pl.kernel,
    out_type=(
        jax.ShapeDtypeStruct((NC, NACC, D), _f32),
        jax.ShapeDtypeStruct((NW, NACC), _f32),
    ),
    mesh=_MESH,
    compiler_params=_SC_PARAMS,
    scratch_types=[
        pltpu.VMEM((NPAD,), _f32),
        pltpu.VMEM((NPAD,), _f32),
        pltpu.VMEM((CHS,), _i32),
        pltpu.VMEM((CHS,), _i32),
        pltpu.VMEM((CHS,), _i32),
        pltpu.VMEM((CHS,), _i32),
        pltpu.VMEM((CHS, D), _f32),
        pltpu.VMEM((CHS, D), _f32),
        pltpu.VMEM((CHS,), _f32),
        pltpu.VMEM((NACC,), _f32),
        pltpu.VMEM_SHARED((NACC, D), _f32),
        pltpu.SemaphoreType.DMA,
        pltpu.SemaphoreType.DMA,
    ],
)
def _sc_gat(src_hbm, dst_hbm, tab_hbm, as_hbm, ad_hbm, num_hbm, den_hbm,
            asv, adv, sidx0, didx0, sidx1, didx1, rows0, rows1, eeb, denh,
            acc, g0, g1):
    cid = lax.axis_index("c")
    sid = lax.axis_index("s")
    wid = cid * NS + sid
    zeros16 = jnp.zeros((16,), _f32)

    pltpu.sync_copy(as_hbm, asv)
    pltpu.sync_copy(ad_hbm, adv)

    def zero_hist(i, c):
        denh[pl.ds(i * 16, 16)] = zeros16
        return c

    lax.fori_loop(0, NACC // 16, zero_hist, 0)

    def zero_body(i, c):
        for j in range(D // 16):
            rows0[i, pl.ds(j * 16, 16)] = zeros16
        return c

    lax.fori_loop(0, CHS, zero_body, 0)
    _zero_acc(rows0, acc, sid, CHS)
    plsc.subcore_barrier()

    base = sid * TEP + cid * TE0
    nch = jnp.where(cid == 0, TE0 // CHS, TE1 // CHS)

    def load_idx(si, di, c):
        eb = base + c * CHS
        pltpu.sync_copy(src_hbm.at[pl.ds(eb, CHS)], si)
        pltpu.sync_copy(dst_hbm.at[pl.ds(eb, CHS)], di)

    def compute_scatter(si, di, rows):
        for g in range(CHS // 16):
            sv = si[pl.ds(g * 16, 16)]
            dv = di[pl.ds(g * 16, 16)]
            e = plsc.load_gather(asv, [sv]) + plsc.load_gather(adv, [dv])
            e = jnp.where(e > 0.0, e, e * 0.2)
            ee = jnp.exp(e)
            eeb[pl.ds(g * 16, 16)] = ee
            plsc.addupdate_scatter(denh, [dv], ee)

        def row(i, c2):
            s = plsc.load_gather(eeb, [jnp.zeros((16,), _i32) + i])
            for j in range(D // 16):
                rows[i, pl.ds(j * 16, 16)] = rows[i, pl.ds(j * 16, 16)] * s
            return c2

        lax.fori_loop(0, CHS, row, 0)
        pltpu.sync_copy(rows, acc.at[di], add=True)

    load_idx(sidx0, didx0, 0)
    pltpu.async_copy(tab_hbm.at[sidx0], rows0, g0)

    def body(cc, carry):
        c1 = 2 * cc + 1
        load_idx(sidx1, didx1, c1)
        pltpu.async_copy(tab_hbm.at[sidx1], rows1, g1)
        pltpu.make_async_copy(tab_hbm.at[pl.ds(0, CHS)], rows0, g0).wait()
        compute_scatter(sidx0, didx0, rows0)

        @pl.when(2 * cc + 2 < nch)
        def _():
            load_idx(sidx0, didx0, 2 * cc + 2)
            pltpu.async_copy(tab_hbm.at[sidx0], rows0, g0)

        pltpu.make_async_copy(tab_hbm.at[pl.ds(0, CHS)], rows1, g1).wait()
        compute_scatter(sidx1, didx1, rows1)
        return carry

    lax.fori_loop(0, nch // 2, body, 0)
    plsc.subcore_barrier()
    pltpu.sync_copy(acc.at[pl.ds(sid * RPT, RPT)],
                    num_hbm.at[cid, pl.ds(sid * RPT, RPT)])
    pltpu.sync_copy(denh, den_hbm.at[wid])


@functools.partial(
    pl.kernel,
    out_type=jax.ShapeDtypeStruct((NC, NACC, D), _f32),
    mesh=_MESH,
    compiler_params=_SC_PARAMS,
    scratch_types=[
        pltpu.VMEM((CHS,), _i32),
        pltpu.VMEM((CHS,), _i32),
        pltpu.VMEM((CHS,), _i32),
        pltpu.VMEM((CHS,), _i32),
        pltpu.VMEM((CHS, D), _f32),
        pltpu.VMEM((CHS, D), _f32),
        pltpu.VMEM((CHS, D), _f32),
        pltpu.VMEM((CHS, D), _f32),
        pltpu.VMEM((CHS, D), _f32),
        pltpu.VMEM((CHS, D), _f32),
        pltpu.VMEM_SHARED((NACC, D), _f32),
        pltpu.SemaphoreType.DMA,
        pltpu.SemaphoreType.DMA,
    ],
)
def _sc_rg(src_hbm, dst_hbm, k_hbm, q_hbm, v_hbm, out_hbm,
           sidx0, didx0, sidx1, didx1, kr0, qr0, vr0, kr1, qr1, vr1,
           acc, g0, g1):
    cid = lax.axis_index("c")
    sid = lax.axis_index("s")
    zeros16 = jnp.zeros((16,), _f32)
    ones16 = jnp.ones((16,), _f32)

    def zero_body(i, c):
        for j in range(D // 16):
            kr0[i, pl.ds(j * 16, 16)] = zeros16
        return c

    lax.fori_loop(0, CHS, zero_body, 0)
    _zero_acc(kr0, acc, sid, CHS)
    plsc.subcore_barrier()

    base = sid * TEP + cid * TE0
    nch = jnp.where(cid == 0, TE0 // CHS, TE1 // CHS)

    def load_idx(si, di, c):
        eb = base + c * CHS
        pltpu.sync_copy(src_hbm.at[pl.ds(eb, CHS)], si)
        pltpu.sync_copy(dst_hbm.at[pl.ds(eb, CHS)], di)

    def issue(si, di, kb, qb, vb, sem):
        pltpu.async_copy(k_hbm.at[di], kb, sem)
        pltpu.async_copy(q_hbm.at[si], qb, sem)
        pltpu.async_copy(v_hbm.at[si], vb, sem)

    def drain(kb, qb, vb, sem):
        pltpu.make_async_copy(k_hbm.at[pl.ds(0, CHS)], kb, sem).wait()
        pltpu.make_async_copy(q_hbm.at[pl.ds(0, CHS)], qb, sem).wait()
        pltpu.make_async_copy(v_hbm.at[pl.ds(0, CHS)], vb, sem).wait()

    def compute_scatter(kb, qb, vb, di):
        def row(i, c2):
            for j in range(D // 16):
                sl = pl.ds(j * 16, 16)
                t = kb[i, sl] + qb[i, sl]
                kb[i, sl] = vb[i, sl] / (ones16 + jnp.exp(-t))
            return c2

        lax.fori_loop(0, CHS, row, 0)
        pltpu.sync_copy(kb, acc.at[di], add=True)

    load_idx(sidx0, didx0, 0)
    issue(sidx0, didx0, kr0, qr0, vr0, g0)

    def body(cc, carry):
        c1 = 2 * cc + 1
        load_idx(sidx1, didx1, c1)
        issue(sidx1, didx1, kr1, qr1, vr1, g1)
        drain(kr0, qr0, vr0, g0)
        compute_scatter(kr0, qr0, vr0, didx0)

        @pl.when(2 * cc + 2 < nch)
        def _():
            load_idx(sidx0, didx0, 2 * cc + 2)
            issue(sidx0, didx0, kr0, qr0, vr0, g0)

        drain(kr1, qr1, vr1, g1)
        compute_scatter(kr1, qr1, vr1, didx1)
        return carry

    lax.fori_loop(0, nch // 2, body, 0)
    plsc.subcore_barrier()
    pltpu.sync_copy(acc.at[pl.ds(sid * RPT, RPT)],
                    out_hbm.at[cid, pl.ds(sid * RPT, RPT)])


def _bn_rows(h, g, b):
    m = jnp.mean(h, axis=0, keepdims=True)
    v = jnp.mean((h - m) * (h - m), axis=0, keepdims=True)
    return (h - m) / jnp.sqrt(v + 1e-5) * g + b


def _dot(a, b):
    return jnp.dot(a, b, preferred_element_type=_f32)


def _tc_enc_body(degp, x, eW, eb, eg, ebe, gW, hlp, selfc, dinv_o):
    deg = lax.dot_general(degp[...], jnp.ones((NW, 1), _f32),
                          (((0,), (0,)), ((), ())))[:N] + 1.0
    dinv = lax.rsqrt(deg)
    h = _dot(x[...], eW[...]) + eb[...]
    h = _bn_rows(h, eg[...], ebe[...])
    h = jnp.maximum(h, 0.0)
    hl = _dot(h, gW[...])
    hlp[0:N, :] = hl * dinv
    hlp[N:NPAD, :] = jnp.zeros((NPAD - N, D), _f32)
    selfc[...] = hl * (dinv * dinv)
    dinv_o[...] = dinv


_tc_enc = pl.pallas_call(
    _tc_enc_body,
    out_shape=[
        jax.ShapeDtypeStruct((NPAD, D), _f32),
        jax.ShapeDtypeStruct((N, D), _f32),
        jax.ShapeDtypeStruct((N, 1), _f32),
    ],
)


def _tc_gat_prep_body(p, selfc, dinv, gb, g1, b1, gW, gas, gad,
                      hgp, asp, adp, x1_o, snum, sden):
    agg = (p[0, 0:N, :] + p[1, 0:N, :]) * dinv[...] + selfc[...] + gb[...]
    x1 = jnp.maximum(agg, 0.0)
    x1 = _bn_rows(x1, g1[...], b1[...])
    hg = _dot(x1, gW[...])
    a_s = _dot(hg, gas[...])
    a_d = _dot(hg, gad[...])
    es = a_s + a_d
    es = jnp.where(es > 0.0, es, es * 0.2)
    ee = jnp.exp(es)
    hgp[0:N, :] = hg
    hgp[N:NPAD, :] = jnp.zeros((NPAD - N, D), _f32)
    asp[0:N, :] = a_s
    asp[N:NPAD, :] = jnp.zeros((NPAD - N, 1), _f32)
    adp[0:N, :] = a_d
    adp[N:NPAD, :] = jnp.zeros((NPAD - N, 1), _f32)
    x1_o[...] = x1
    snum[...] = hg * ee
    sden[...] = ee


_tc_gat_prep = pl.pallas_call(
    _tc_gat_prep_body,
    out_shape=[
        jax.ShapeDtypeStruct((NPAD, D), _f32),
        jax.ShapeDtypeStruct((NPAD, 1), _f32),
        jax.ShapeDtypeStruct((NPAD, 1), _f32),
        jax.ShapeDtypeStruct((N, D), _f32),
        jax.ShapeDtypeStruct((N, D), _f32),
        jax.ShapeDtypeStruct((N, 1), _f32),
    ],
)


def _tc_rg_prep_body(p, denp, snum, sden, x1, gatb, g2, b2, Wk, Wq, Wv, Ws,
                     kp, qp, vp, skip_o, x2_o):
    num = p[0, 0:N, :] + p[1, 0:N, :] + snum[...]
    den = lax.dot_general(denp[...], jnp.ones((NW, 1), _f32),
                          (((0,), (0,)), ((), ())))[:N] + sden[...]
    x2 = num / (den + 1e-16) + gatb[...]
    x2 = jnp.maximum(x2, 0.0)
    x2 = _bn_rows(x2, g2[...], b2[...])
    x2 = x1[...] + x2
    zpad = jnp.zeros((NPAD - N, D), _f32)
    kp[0:N, :] = _dot(x2, Wk[...])
    kp[N:NPAD, :] = zpad
    qp[0:N, :] = _dot(x2, Wq[...])
    qp[N:NPAD, :] = zpad
    vp[0:N, :] = _dot(x2, Wv[...])
    vp[N:NPAD, :] = zpad
    skip_o[...] = _dot(x2, Ws[...])
    x2_o[...] = x2


_tc_rg_prep = pl.pallas_call(
    _tc_rg_prep_body,
    out_shape=[
        jax.ShapeDtypeStruct((NPAD, D), _f32),
        jax.ShapeDtypeStruct((NPAD, D), _f32),
        jax.ShapeDtypeStruct((NPAD, D), _f32),
        jax.ShapeDtypeStruct((N, D), _f32),
        jax.ShapeDtypeStruct((N, D), _f32),
    ],
)


def _tc_cls_body(p, skip, x2, rgb, g3, b3, W1, c1, W2, c2, out):
    x3 = p[0, 0:N, :] + p[1, 0:N, :] + skip[...] + rgb[...]
    x3 = jnp.maximum(x3, 0.0)
    x3 = _bn_rows(x3, g3[...], b3[...])
    x3 = x2[...] + x3
    z = jnp.maximum(_dot(x3, W1[...]) + c1[...], 0.0)
    out[...] = _dot(z, W2[...]) + c2[...]


_tc_cls = pl.pallas_call(
    _tc_cls_body,
    out_shape=jax.ShapeDtypeStruct((N, C_OUT), _f32),
)


def kernel(x, edge_index, enc_W, enc_b, enc_g, enc_be, gcn_W, gcn_b,
           gat_W, gat_as, gat_ad, gat_b, rg_Wk, rg_Wq, rg_Wv, rg_Ws, rg_b,
           bn1_g, bn1_b, bn2_g, bn2_b, bn3_g, bn3_b,
           cls_W1, cls_b1, cls_W2, cls_b2):
    pad = jnp.full((EPADX - E,), N, _i32)
    srcp = jnp.concatenate([edge_index[0].astype(_i32), pad])
    dstp = jnp.concatenate([edge_index[1].astype(_i32), pad])

    degp = _sc_deg(dstp)

    hlp, selfc, dinv = _tc_enc(
        degp, x, enc_W, enc_b.reshape(1, D), enc_g.reshape(1, D),
        enc_be.reshape(1, D), gcn_W)

    gcn_parts = _sc_gcn(srcp, dstp, hlp)

    hgp, asp, adp, x1, snum, sden = _tc_gat_prep(
        gcn_parts, selfc, dinv, gcn_b.reshape(1, D), bn1_g.reshape(1, D),
        bn1_b.reshape(1, D), gat_W, gat_as.reshape(D, 1), gat_ad.reshape(D, 1))

    gat_num, gat_den = _sc_gat(srcp, dstp, hgp, asp.reshape(NPAD),
                               adp.reshape(NPAD))

    kp, qp, vp, skip, x2 = _tc_rg_prep(
        gat_num, gat_den, snum, sden, x1, gat_b.reshape(1, D),
        bn2_g.reshape(1, D), bn2_b.reshape(1, D), rg_Wk, rg_Wq, rg_Wv, rg_Ws)

    rg_parts = _sc_rg(srcp, dstp, kp, qp, vp)

    logits = _tc_cls(
        rg_parts, skip, x2, rg_b.reshape(1, D), bn3_g.reshape(1, D),
        bn3_b.reshape(1, D), cls_W1, cls_b1.reshape(1, HID), cls_W2,
        cls_b2.reshape(1, C_OUT))
    return logits


# speedup vs baseline: 14.1747x; 1.0568x over previous
import functools

import jax
import jax.numpy as jnp
from jax import lax
from jax.experimental import pallas as pl
from jax.experimental.pallas import tpu as pltpu
from jax.experimental.pallas import tpu_sc as plsc

N = 10000
D = 128
E = 320000
HID = 64
C_OUT = 21

NC = 2
NS = 16
NW = NC * NS
NPAD = 10240
EPAD = 327680
TEP = EPAD // NS
TE0 = 13824
TE1 = TEP - TE0
CH = 128
CHS = 64
NACC = 10112
RPT = NACC // NS
EPADX = EPAD + 2 * CH

_f32 = jnp.float32
_i32 = jnp.int32

_MESH = plsc.VectorSubcoreMesh(core_axis_name="c", subcore_axis_name="s",
                               num_cores=NC, num_subcores=NS)
_SC_PARAMS = pltpu.CompilerParams(needs_layout_passes=False)


def _zero_acc(zbuf, acc, sid, nrows):
    off = sid * RPT
    done = 0
    while done < RPT:
        step = min(nrows, RPT - done)
        pltpu.sync_copy(zbuf.at[pl.ds(0, step)],
                        acc.at[pl.ds(off + done, step)])
        done += step


@functools.partial(
    pl.kernel,
    out_type=jax.ShapeDtypeStruct((NW, NACC), _f32),
    mesh=_MESH,
    compiler_params=_SC_PARAMS,
    scratch_types=[
        pltpu.VMEM((TE0,), _i32),
        pltpu.VMEM((NACC,), _f32),
    ],
)
def _sc_deg(dst_hbm, out_hbm, idxv, hist):
    cid = lax.axis_index("c")
    sid = lax.axis_index("s")
    wid = cid * NS + sid
    base = sid * TEP + cid * TE0

    @pl.when(cid == 0)
    def _():
        pltpu.sync_copy(dst_hbm.at[pl.ds(base, TE0)], idxv.at[pl.ds(0, TE0)])

    @pl.when(cid == 1)
    def _():
        pltpu.sync_copy(dst_hbm.at[pl.ds(base, TE1)], idxv.at[pl.ds(0, TE1)])

    zeros16 = jnp.zeros((16,), _f32)
    ones16 = jnp.ones((16,), _f32)

    def zero_body(i, c):
        hist[pl.ds(i * 16, 16)] = zeros16
        return c

    lax.fori_loop(0, NACC // 16, zero_body, 0)

    def body(g, c):
        di = idxv[pl.ds(g * 16, 16)]
        plsc.addupdate_scatter(hist, [di], ones16)
        return c

    n16 = jnp.where(cid == 0, TE0 // 16, TE1 // 16)
    lax.fori_loop(0, n16, body, 0)
    pltpu.sync_copy(hist, out_hbm.at[wid])


@functools.partial(
    pl.kernel,
    out_type=jax.ShapeDtypeStruct((NC, NACC, D), _f32),
    mesh=_MESH,
    compiler_params=_SC_PARAMS,
    scratch_types=[
        pltpu.VMEM((CH,), _i32),
        pltpu.VMEM((CH,), _i32),
        pltpu.VMEM((CH,), _i32),
        pltpu.VMEM((CH,), _i32),
        pltpu.VMEM((CH, D), _f32),
        pltpu.VMEM((CH, D), _f32),
        pltpu.VMEM_SHARED((NACC, D), _f32),
        pltpu.SemaphoreType.DMA,
        pltpu.SemaphoreType.DMA,
        pltpu.SemaphoreType.DMA,
        pltpu.SemaphoreType.DMA,
    ],
)
def _sc_gcn(src_hbm, dst_hbm, tab_hbm, out_hbm,
            sidx0, didx0, sidx1, didx1, rows0, rows1, acc, g0, g1, i0, i1):
    cid = lax.axis_index("c")
    sid = lax.axis_index("s")
    zeros16 = jnp.zeros((16,), _f32)

    def zero_body(i, c):
        for j in range(D // 16):
            rows0[i, pl.ds(j * 16, 16)] = zeros16
        return c

    lax.fori_loop(0, CH, zero_body, 0)
    _zero_acc(rows0, acc, sid, CH)
    plsc.subcore_barrier()

    base = sid * TEP + cid * TE0
    nch = jnp.where(cid == 0, TE0 // CH, TE1 // CH)

    def issue_idx(si, di, c, sem):
        eb = base + c * CH
        pltpu.async_copy(src_hbm.at[pl.ds(eb, CH)], si, sem)
        pltpu.async_copy(dst_hbm.at[pl.ds(eb, CH)], di, sem)

    def drain_idx(si, di, sem):
        pltpu.make_async_copy(src_hbm.at[pl.ds(0, CH)], si, sem).wait()
        pltpu.make_async_copy(dst_hbm.at[pl.ds(0, CH)], di, sem).wait()

    issue_idx(sidx0, didx0, 0, i0)
    drain_idx(sidx0, didx0, i0)
    pltpu.async_copy(tab_hbm.at[sidx0], rows0, g0)
    issue_idx(sidx1, didx1, 1, i1)

    def body(cc, carry):
        has_next = 2 * cc + 2 < nch
        drain_idx(sidx1, didx1, i1)
        pltpu.async_copy(tab_hbm.at[sidx1], rows1, g1)
        pltpu.make_async_copy(tab_hbm.at[pl.ds(0, CH)], rows0, g0).wait()
        pltpu.sync_copy(rows0, acc.at[didx0], add=True)

        @pl.when(has_next)
        def _():
            issue_idx(sidx0, didx0, 2 * cc + 2, i0)

        pltpu.make_async_copy(tab_hbm.at[pl.ds(0, CH)], rows1, g1).wait()

        @pl.when(has_next)
        def _():
            drain_idx(sidx0, didx0, i0)
            pltpu.async_copy(tab_hbm.at[sidx0], rows0, g0)

        pltpu.sync_copy(rows1, acc.at[didx1], add=True)

        @pl.when(has_next)
        def _():
            issue_idx(sidx1, didx1, 2 * cc + 3, i1)

        return carry

    lax.fori_loop(0, nch // 2, body, 0)
    plsc.subcore_barrier()
    pltpu.sync_copy(acc.at[pl.ds(sid * RPT, RPT)],
                    out_hbm.at[cid, pl.ds(sid * RPT, RPT)])


@functools.partial(
    pl.kernel,
    out_type=(
        jax.ShapeDtypeStruct((NC, NACC, D), _f32),
        jax.ShapeDtypeStruct((NW, NACC), _f32),
    ),
    mesh=_MESH,
    compiler_params=_SC_PARAMS,
    scratch_types=[
        pltpu.VMEM((NPAD,), _f32),
        pltpu.VMEM((NPAD,), _f32),
        pltpu.VMEM((CHS,), _i32),
        pltpu.VMEM((CHS,), _i32),
        pltpu.VMEM((CHS,), _i32),
        pltpu.VMEM((CHS,), _i32),
        pltpu.VMEM((CHS, D), _f32),
        pltpu.VMEM((CHS, D), _f32),
        pltpu.VMEM((CHS,), _f32),
        pltpu.VMEM((NACC,), _f32),
        pltpu.VMEM_SHARED((NACC, D), _f32),
        pltpu.SemaphoreType.DMA,
        pltpu.SemaphoreType.DMA,
        pltpu.SemaphoreType.DMA,
        pltpu.SemaphoreType.DMA,
    ],
)
def _sc_gat(src_hbm, dst_hbm, tab_hbm, as_hbm, ad_hbm, num_hbm, den_hbm,
            asv, adv, sidx0, didx0, sidx1, didx1, rows0, rows1, eeb, denh,
            acc, g0, g1, i0, i1):
    cid = lax.axis_index("c")
    sid = lax.axis_index("s")
    wid = cid * NS + sid
    zeros16 = jnp.zeros((16,), _f32)

    pltpu.sync_copy(as_hbm, asv)
    pltpu.sync_copy(ad_hbm, adv)

    def zero_hist(i, c):
        denh[pl.ds(i * 16, 16)] = zeros16
        return c

    lax.fori_loop(0, NACC // 16, zero_hist, 0)

    def zero_body(i, c):
        for j in range(D // 16):
            rows0[i, pl.ds(j * 16, 16)] = zeros16
        return c

    lax.fori_loop(0, CHS, zero_body, 0)
    _zero_acc(rows0, acc, sid, CHS)
    plsc.subcore_barrier()

    base = sid * TEP + cid * TE0
    nch = jnp.where(cid == 0, TE0 // CHS, TE1 // CHS)

    def issue_idx(si, di, c, sem):
        eb = base + c * CHS
        pltpu.async_copy(src_hbm.at[pl.ds(eb, CHS)], si, sem)
        pltpu.async_copy(dst_hbm.at[pl.ds(eb, CHS)], di, sem)

    def drain_idx(si, di, sem):
        pltpu.make_async_copy(src_hbm.at[pl.ds(0, CHS)], si, sem).wait()
        pltpu.make_async_copy(dst_hbm.at[pl.ds(0, CHS)], di, sem).wait()

    def compute_scatter(si, di, rows):
        for g in range(CHS // 16):
            sv = si[pl.ds(g * 16, 16)]
            dv = di[pl.ds(g * 16, 16)]
            e = plsc.load_gather(asv, [sv]) + plsc.load_gather(adv, [dv])
            e = jnp.where(e > 0.0, e, e * 0.2)
            ee = jnp.exp(e)
            eeb[pl.ds(g * 16, 16)] = ee
            plsc.addupdate_scatter(denh, [dv], ee)

        def row(i, c2):
            s = plsc.load_gather(eeb, [jnp.zeros((16,), _i32) + i])
            for j in range(D // 16):
                rows[i, pl.ds(j * 16, 16)] = rows[i, pl.ds(j * 16, 16)] * s
            return c2

        lax.fori_loop(0, CHS, row, 0)
        pltpu.sync_copy(rows, acc.at[di], add=True)

    issue_idx(sidx0, didx0, 0, i0)
    drain_idx(sidx0, didx0, i0)
    pltpu.async_copy(tab_hbm.at[sidx0], rows0, g0)
    issue_idx(sidx1, didx1, 1, i1)

    def body(cc, carry):
        has_next = 2 * cc + 2 < nch
        drain_idx(sidx1, didx1, i1)
        pltpu.async_copy(tab_hbm.at[sidx1], rows1, g1)
        pltpu.make_async_copy(tab_hbm.at[pl.ds(0, CHS)], rows0, g0).wait()
        compute_scatter(sidx0, didx0, rows0)

        @pl.when(has_next)
        def _():
            issue_idx(sidx0, didx0, 2 * cc + 2, i0)

        pltpu.make_async_copy(tab_hbm.at[pl.ds(0, CHS)], rows1, g1).wait()

        @pl.when(has_next)
        def _():
            drain_idx(sidx0, didx0, i0)
            pltpu.async_copy(tab_hbm.at[sidx0], rows0, g0)

        compute_scatter(sidx1, didx1, rows1)

        @pl.when(has_next)
        def _():
            issue_idx(sidx1, didx1, 2 * cc + 3, i1)

        return carry

    lax.fori_loop(0, nch // 2, body, 0)
    plsc.subcore_barrier()
    pltpu.sync_copy(acc.at[pl.ds(sid * RPT, RPT)],
                    num_hbm.at[cid, pl.ds(sid * RPT, RPT)])
    pltpu.sync_copy(denh, den_hbm.at[wid])


@functools.partial(
    pl.kernel,
    out_type=jax.ShapeDtypeStruct((NC, NACC, D), _f32),
    mesh=_MESH,
    compiler_params=_SC_PARAMS,
    scratch_types=[
        pltpu.VMEM((CHS,), _i32),
        pltpu.VMEM((CHS,), _i32),
        pltpu.VMEM((CHS,), _i32),
        pltpu.VMEM((CHS,), _i32),
        pltpu.VMEM((CHS, D), _f32),
        pltpu.VMEM((CHS, D), _f32),
        pltpu.VMEM((CHS, D), _f32),
        pltpu.VMEM((CHS, D), _f32),
        pltpu.VMEM((CHS, D), _f32),
        pltpu.VMEM((CHS, D), _f32),
        pltpu.VMEM_SHARED((NACC, D), _f32),
        pltpu.SemaphoreType.DMA,
        pltpu.SemaphoreType.DMA,
        pltpu.SemaphoreType.DMA,
        pltpu.SemaphoreType.DMA,
    ],
)
def _sc_rg(src_hbm, dst_hbm, k_hbm, q_hbm, v_hbm, out_hbm,
           sidx0, didx0, sidx1, didx1, kr0, qr0, vr0, kr1, qr1, vr1,
           acc, g0, g1, i0, i1):
    cid = lax.axis_index("c")
    sid = lax.axis_index("s")
    zeros16 = jnp.zeros((16,), _f32)
    ones16 = jnp.ones((16,), _f32)

    def zero_body(i, c):
        for j in range(D // 16):
            kr0[i, pl.ds(j * 16, 16)] = zeros16
        return c

    lax.fori_loop(0, CHS, zero_body, 0)
    _zero_acc(kr0, acc, sid, CHS)
    plsc.subcore_barrier()

    base = sid * TEP + cid * TE0
    nch = jnp.where(cid == 0, TE0 // CHS, TE1 // CHS)

    def issue_idx(si, di, c, sem):
        eb = base + c * CHS
        pltpu.async_copy(src_hbm.at[pl.ds(eb, CHS)], si, sem)
        pltpu.async_copy(dst_hbm.at[pl.ds(eb, CHS)], di, sem)

    def drain_idx(si, di, sem):
        pltpu.make_async_copy(src_hbm.at[pl.ds(0, CHS)], si, sem).wait()
        pltpu.make_async_copy(dst_hbm.at[pl.ds(0, CHS)], di, sem).wait()

    def issue(si, di, kb, qb, vb, sem):
        pltpu.async_copy(k_hbm.at[di], kb, sem)
        pltpu.async_copy(q_hbm.at[si], qb, sem)
        pltpu.async_copy(v_hbm.at[si], vb, sem)

    def drain(kb, qb, vb, sem):
        pltpu.make_async_copy(k_hbm.at[pl.ds(0, CHS)], kb, sem).wait()
        pltpu.make_async_copy(q_hbm.at[pl.ds(0, CHS)], qb, sem).wait()
        pltpu.make_async_copy(v_hbm.at[pl.ds(0, CHS)], vb, sem).wait()

    def compute_scatter(kb, qb, vb, di):
        def row(i, c2):
            for j in range(D // 16):
                sl = pl.ds(j * 16, 16)
                t = kb[i, sl] + qb[i, sl]
                kb[i, sl] = vb[i, sl] / (ones16 + jnp.exp(-t))
            return c2

        lax.fori_loop(0, CHS, row, 0)
        pltpu.sync_copy(kb, acc.at[di], add=True)

    issue_idx(sidx0, didx0, 0, i0)
    drain_idx(sidx0, didx0, i0)
    issue(sidx0, didx0, kr0, qr0, vr0, g0)
    issue_idx(sidx1, didx1, 1, i1)

    def body(cc, carry):
        has_next = 2 * cc + 2 < nch
        drain_idx(sidx1, didx1, i1)
        issue(sidx1, didx1, kr1, qr1, vr1, g1)
        drain(kr0, qr0, vr0, g0)
        compute_scatter(kr0, qr0, vr0, didx0)

        @pl.when(has_next)
        def _():
            issue_idx(sidx0, didx0, 2 * cc + 2, i0)

        drain(kr1, qr1, vr1, g1)

        @pl.when(has_next)
        def _():
            drain_idx(sidx0, didx0, i0)
            issue(sidx0, didx0, kr0, qr0, vr0, g0)

        compute_scatter(kr1, qr1, vr1, didx1)

        @pl.when(has_next)
        def _():
            issue_idx(sidx1, didx1, 2 * cc + 3, i1)

        return carry

    lax.fori_loop(0, nch // 2, body, 0)
    plsc.subcore_barrier()
    pltpu.sync_copy(acc.at[pl.ds(sid * RPT, RPT)],
                    out_hbm.at[cid, pl.ds(sid * RPT, RPT)])


def _bn_rows(h, g, b):
    m = jnp.mean(h, axis=0, keepdims=True)
    v = jnp.mean((h - m) * (h - m), axis=0, keepdims=True)
    return (h - m) / jnp.sqrt(v + 1e-5) * g + b


def _dot(a, b):
    return jnp.dot(a, b, preferred_element_type=_f32)


def _tc_enc_body(degp, x, eW, eb, eg, ebe, gW, hlp, selfc, dinv_o):
    deg = lax.dot_general(degp[...], jnp.ones((NW, 1), _f32),
                          (((0,), (0,)), ((), ())))[:N] + 1.0
    dinv = lax.rsqrt(deg)
    h = _dot(x[...], eW[...]) + eb[...]
    h = _bn_rows(h, eg[...], ebe[...])
    h = jnp.maximum(h, 0.0)
    hl = _dot(h, gW[...])
    hlp[0:N, :] = hl * dinv
    hlp[N:NPAD, :] = jnp.zeros((NPAD - N, D), _f32)
    selfc[...] = hl * (dinv * dinv)
    dinv_o[...] = dinv


_tc_enc = pl.pallas_call(
    _tc_enc_body,
    out_shape=[
        jax.ShapeDtypeStruct((NPAD, D), _f32),
        jax.ShapeDtypeStruct((N, D), _f32),
        jax.ShapeDtypeStruct((N, 1), _f32),
    ],
)


def _tc_gat_prep_body(p, selfc, dinv, gb, g1, b1, gW, gas, gad,
                      hgp, asp, adp, x1_o, snum, sden):
    agg = (p[0, 0:N, :] + p[1, 0:N, :]) * dinv[...] + selfc[...] + gb[...]
    x1 = jnp.maximum(agg, 0.0)
    x1 = _bn_rows(x1, g1[...], b1[...])
    hg = _dot(x1, gW[...])
    a_s = _dot(hg, gas[...])
    a_d = _dot(hg, gad[...])
    es = a_s + a_d
    es = jnp.where(es > 0.0, es, es * 0.2)
    ee = jnp.exp(es)
    hgp[0:N, :] = hg
    hgp[N:NPAD, :] = jnp.zeros((NPAD - N, D), _f32)
    asp[0:N, :] = a_s
    asp[N:NPAD, :] = jnp.zeros((NPAD - N, 1), _f32)
    adp[0:N, :] = a_d
    adp[N:NPAD, :] = jnp.zeros((NPAD - N, 1), _f32)
    x1_o[...] = x1
    snum[...] = hg * ee
    sden[...] = ee


_tc_gat_prep = pl.pallas_call(
    _tc_gat_prep_body,
    out_shape=[
        jax.ShapeDtypeStruct((NPAD, D), _f32),
        jax.ShapeDtypeStruct((NPAD, 1), _f32),
        jax.ShapeDtypeStruct((NPAD, 1), _f32),
        jax.ShapeDtypeStruct((N, D), _f32),
        jax.ShapeDtypeStruct((N, D), _f32),
        jax.ShapeDtypeStruct((N, 1), _f32),
    ],
)


def _tc_rg_prep_body(p, denp, snum, sden, x1, gatb, g2, b2, Wk, Wq, Wv, Ws,
                     kp, qp, vp, skip_o, x2_o):
    num = p[0, 0:N, :] + p[1, 0:N, :] + snum[...]
    den = lax.dot_general(denp[...], jnp.ones((NW, 1), _f32),
                          (((0,), (0,)), ((), ())))[:N] + sden[...]
    x2 = num / (den + 1e-16) + gatb[...]
    x2 = jnp.maximum(x2, 0.0)
    x2 = _bn_rows(x2, g2[...], b2[...])
    x2 = x1[...] + x2
    zpad = jnp.zeros((NPAD - N, D), _f32)
    kp[0:N, :] = _dot(x2, Wk[...])
    kp[N:NPAD, :] = zpad
    qp[0:N, :] = _dot(x2, Wq[...])
    qp[N:NPAD, :] = zpad
    vp[0:N, :] = _dot(x2, Wv[...])
    vp[N:NPAD, :] = zpad
    skip_o[...] = _dot(x2, Ws[...])
    x2_o[...] = x2


_tc_rg_prep = pl.pallas_call(
    _tc_rg_prep_body,
    out_shape=[
        jax.ShapeDtypeStruct((NPAD, D), _f32),
        jax.ShapeDtypeStruct((NPAD, D), _f32),
        jax.ShapeDtypeStruct((NPAD, D), _f32),
        jax.ShapeDtypeStruct((N, D), _f32),
        jax.ShapeDtypeStruct((N, D), _f32),
    ],
)


def _tc_cls_body(p, skip, x2, rgb, g3, b3, W1, c1, W2, c2, out):
    x3 = p[0, 0:N, :] + p[1, 0:N, :] + skip[...] + rgb[...]
    x3 = jnp.maximum(x3, 0.0)
    x3 = _bn_rows(x3, g3[...], b3[...])
    x3 = x2[...] + x3
    z = jnp.maximum(_dot(x3, W1[...]) + c1[...], 0.0)
    out[...] = _dot(z, W2[...]) + c2[...]


_tc_cls = pl.pallas_call(
    _tc_cls_body,
    out_shape=jax.ShapeDtypeStruct((N, C_OUT), _f32),
)


def kernel(x, edge_index, enc_W, enc_b, enc_g, enc_be, gcn_W, gcn_b,
           gat_W, gat_as, gat_ad, gat_b, rg_Wk, rg_Wq, rg_Wv, rg_Ws, rg_b,
           bn1_g, bn1_b, bn2_g, bn2_b, bn3_g, bn3_b,
           cls_W1, cls_b1, cls_W2, cls_b2):
    pad = jnp.full((EPADX - E,), N, _i32)
    srcp = jnp.concatenate([edge_index[0].astype(_i32), pad])
    dstp = jnp.concatenate([edge_index[1].astype(_i32), pad])

    degp = _sc_deg(dstp)

    hlp, selfc, dinv = _tc_enc(
        degp, x, enc_W, enc_b.reshape(1, D), enc_g.reshape(1, D),
        enc_be.reshape(1, D), gcn_W)

    gcn_parts = _sc_gcn(srcp, dstp, hlp)

    hgp, asp, adp, x1, snum, sden = _tc_gat_prep(
        gcn_parts, selfc, dinv, gcn_b.reshape(1, D), bn1_g.reshape(1, D),
        bn1_b.reshape(1, D), gat_W, gat_as.reshape(D, 1), gat_ad.reshape(D, 1))

    gat_num, gat_den = _sc_gat(srcp, dstp, hgp, asp.reshape(NPAD),
                               adp.reshape(NPAD))

    kp, qp, vp, skip, x2 = _tc_rg_prep(
        gat_num, gat_den, snum, sden, x1, gat_b.reshape(1, D),
        bn2_g.reshape(1, D), bn2_b.reshape(1, D), rg_Wk, rg_Wq, rg_Wv, rg_Ws)

    rg_parts = _sc_rg(srcp, dstp, kp, qp, vp)

    logits = _tc_cls(
        rg_parts, skip, x2, rg_b.reshape(1, D), bn3_g.reshape(1, D),
        bn3_b.reshape(1, D), cls_W1, cls_b1.reshape(1, HID), cls_W2,
        cls_b2.reshape(1, C_OUT))
    return logits
